# baseline (speedup 1.0000x reference)
import math
import functools

import jax
import jax.numpy as jnp
from jax.experimental import pallas as pl
from jax.experimental.pallas import tpu as pltpu

_MIN_SIGMA = 0.1
_LOG2E = 1.4426950408889634
_BS = 2048
_CHUNK = 512


def _histogram_kernel(t_ref, st_ref, o_ref, sig_ref, a_ref, *, nv, ns, cn, cf):
    b = pl.program_id(0)
    i = pl.program_id(1)
    nsb = ns // _BS

    @pl.when(i == 0)
    def _():
        t = t_ref[0]
        m = jnp.mean(t, axis=0, keepdims=True)
        ctr = t - m
        var = jnp.sum(ctr * ctr, axis=0, keepdims=True) / (nv - 1)
        sig = jnp.maximum(jnp.mean(jnp.sqrt(var)), _MIN_SIGMA)
        sig_ref[0, 0] = sig
        g = (cn * _LOG2E) / (sig * sig)
        t2 = jnp.sum(t * t, axis=1, keepdims=True)
        a_ref[:, 0:3] = (-2.0 * g) * t
        a_ref[:, 3:4] = g * t2
        a_ref[:, 4:5] = jnp.ones((nv, 1), jnp.float32)

    sig = sig_ref[0, 0]
    g = (cn * _LOG2E) / (sig * sig)

    st = st_ref[...]
    s2 = jnp.sum(st * st, axis=0, keepdims=True)
    as2 = g * s2
    as2_bf = as2.astype(jnp.bfloat16).astype(jnp.float32)
    delta = as2 - as2_bf
    rhs = jnp.concatenate(
        [st, jnp.ones((1, _BS), jnp.float32), as2_bf], axis=0)

    acc = jnp.zeros((1, _BS), jnp.float32)
    for c in range(nv // _CHUNK):
        ac = a_ref[c * _CHUNK:(c + 1) * _CHUNK, :]
        arg = jnp.dot(ac, rhs, preferred_element_type=jnp.float32)
        e = jnp.exp2(arg)
        acc = acc + jnp.sum(e, axis=0, keepdims=True)

    scale = (cf / nv) / (sig * sig * sig)
    off = pl.multiple_of(i * _BS, _BS)
    o_ref[0, 0:1, pl.ds(off, _BS)] = (scale * jnp.exp2(delta)) * acc

    @pl.when(i == nsb - 1)
    def _():
        row = o_ref[0]
        total = jnp.sum(row, axis=1, keepdims=True)
        denom = jnp.maximum(total, 1e-5)
        o_ref[0] = row * (1.0 / denom)


def kernel(T, S):
    B, nv, _ = T.shape
    ns = S.shape[0]
    d = 3
    c = (4.0 / (d + 2)) ** (1.0 / (d + 4)) * float(nv) ** (-1.0 / (d + 4))
    cf = (2.0 * math.pi) ** (-1.5) / c ** 3
    cn = -1.0 / (2.0 * c ** 2)

    st = S.T

    body = functools.partial(_histogram_kernel, nv=nv, ns=ns, cn=cn, cf=cf)
    out = pl.pallas_call(
        body,
        grid=(B, ns // _BS),
        in_specs=[
            pl.BlockSpec((1, nv, 3), lambda b, i: (b, 0, 0)),
            pl.BlockSpec((3, _BS), lambda b, i: (0, i)),
        ],
        out_specs=pl.BlockSpec((1, 1, ns), lambda b, i: (b, 0, 0)),
        out_shape=jax.ShapeDtypeStruct((B, 1, ns), jnp.float32),
        scratch_shapes=[
            pltpu.SMEM((1, 1), jnp.float32),
            pltpu.VMEM((nv, 5), jnp.float32),
        ],
        compiler_params=pltpu.CompilerParams(
            dimension_semantics=("parallel", "arbitrary"),
            vmem_limit_bytes=56 * 1024 * 1024,
        ),
    )(T, st)
    return out.reshape(B, ns)


# speedup vs baseline: 1.0297x; 1.0297x over previous
import math
import functools

import jax
import jax.numpy as jnp
from jax.experimental import pallas as pl
from jax.experimental.pallas import tpu as pltpu

_MIN_SIGMA = 0.1
_LOG2E = 1.4426950408889634
_BS = 4096
_CHUNK = 256


def _histogram_kernel(t_ref, st_ref, o_ref, sig_ref, a_ref, *, nv, ns, cn, cf):
    b = pl.program_id(0)
    i = pl.program_id(1)
    nsb = ns // _BS

    @pl.when(i == 0)
    def _():
        t = t_ref[0]
        m = jnp.mean(t, axis=0, keepdims=True)
        ctr = t - m
        var = jnp.sum(ctr * ctr, axis=0, keepdims=True) / (nv - 1)
        sig = jnp.maximum(jnp.mean(jnp.sqrt(var)), _MIN_SIGMA)
        sig_ref[0, 0] = sig
        g = (cn * _LOG2E) / (sig * sig)
        t2 = jnp.sum(t * t, axis=1, keepdims=True)
        a_ref[:, 0:3] = (-2.0 * g) * t
        a_ref[:, 3:4] = g * t2
        a_ref[:, 4:5] = jnp.ones((nv, 1), jnp.float32)

    sig = sig_ref[0, 0]
    g = (cn * _LOG2E) / (sig * sig)

    st = st_ref[...]
    s2 = jnp.sum(st * st, axis=0, keepdims=True)
    as2 = g * s2
    as2_bf = as2.astype(jnp.bfloat16).astype(jnp.float32)
    delta = as2 - as2_bf
    rhs = jnp.concatenate(
        [st, jnp.ones((1, _BS), jnp.float32), as2_bf], axis=0)

    acc = jnp.zeros((1, _BS), jnp.float32)
    for c in range(nv // _CHUNK):
        ac = a_ref[c * _CHUNK:(c + 1) * _CHUNK, :]
        arg = jnp.dot(ac, rhs, preferred_element_type=jnp.float32)
        e = jnp.exp2(arg)
        acc = acc + jnp.sum(e, axis=0, keepdims=True)

    scale = (cf / nv) / (sig * sig * sig)
    off = pl.multiple_of(i * _BS, _BS)
    o_ref[0, 0:1, pl.ds(off, _BS)] = (scale * jnp.exp2(delta)) * acc

    @pl.when(i == nsb - 1)
    def _():
        row = o_ref[0]
        total = jnp.sum(row, axis=1, keepdims=True)
        denom = jnp.maximum(total, 1e-5)
        o_ref[0] = row * (1.0 / denom)


def kernel(T, S):
    B, nv, _ = T.shape
    ns = S.shape[0]
    d = 3
    c = (4.0 / (d + 2)) ** (1.0 / (d + 4)) * float(nv) ** (-1.0 / (d + 4))
    cf = (2.0 * math.pi) ** (-1.5) / c ** 3
    cn = -1.0 / (2.0 * c ** 2)

    st = S.T

    body = functools.partial(_histogram_kernel, nv=nv, ns=ns, cn=cn, cf=cf)
    out = pl.pallas_call(
        body,
        grid=(B, ns // _BS),
        in_specs=[
            pl.BlockSpec((1, nv, 3), lambda b, i: (b, 0, 0)),
            pl.BlockSpec((3, _BS), lambda b, i: (0, i)),
        ],
        out_specs=pl.BlockSpec((1, 1, ns), lambda b, i: (b, 0, 0)),
        out_shape=jax.ShapeDtypeStruct((B, 1, ns), jnp.float32),
        scratch_shapes=[
            pltpu.SMEM((1, 1), jnp.float32),
            pltpu.VMEM((nv, 5), jnp.float32),
        ],
        compiler_params=pltpu.CompilerParams(
            dimension_semantics=("parallel", "arbitrary"),
            vmem_limit_bytes=56 * 1024 * 1024,
        ),
    )(T, st)
    return out.reshape(B, ns)


# speedup vs baseline: 1.0307x; 1.0009x over previous
import math
import functools

import jax
import jax.numpy as jnp
from jax.experimental import pallas as pl
from jax.experimental.pallas import tpu as pltpu

_MIN_SIGMA = 0.1
_LOG2E = 1.4426950408889634
_BS = 4096
_CHUNK = 128


def _histogram_kernel(t_ref, st_ref, o_ref, sig_ref, a_ref, *, nv, ns, cn, cf):
    b = pl.program_id(0)
    i = pl.program_id(1)
    nsb = ns // _BS

    @pl.when(i == 0)
    def _():
        t = t_ref[0]
        m = jnp.mean(t, axis=0, keepdims=True)
        ctr = t - m
        var = jnp.sum(ctr * ctr, axis=0, keepdims=True) / (nv - 1)
        sig = jnp.maximum(jnp.mean(jnp.sqrt(var)), _MIN_SIGMA)
        sig_ref[0, 0] = sig
        g = (cn * _LOG2E) / (sig * sig)
        t2 = jnp.sum(t * t, axis=1, keepdims=True)
        a_ref[:, 0:3] = (-2.0 * g) * t
        a_ref[:, 3:4] = g * t2
        a_ref[:, 4:5] = jnp.ones((nv, 1), jnp.float32)

    sig = sig_ref[0, 0]
    g = (cn * _LOG2E) / (sig * sig)

    st = st_ref[...]
    s2 = jnp.sum(st * st, axis=0, keepdims=True)
    as2 = g * s2
    as2_bf = as2.astype(jnp.bfloat16).astype(jnp.float32)
    delta = as2 - as2_bf
    rhs = jnp.concatenate(
        [st, jnp.ones((1, _BS), jnp.float32), as2_bf], axis=0)

    acc = jnp.zeros((1, _BS), jnp.float32)
    for c in range(nv // _CHUNK):
        ac = a_ref[c * _CHUNK:(c + 1) * _CHUNK, :]
        arg = jnp.dot(ac, rhs, preferred_element_type=jnp.float32)
        e = jnp.exp2(arg)
        acc = acc + jnp.sum(e, axis=0, keepdims=True)

    scale = (cf / nv) / (sig * sig * sig)
    off = pl.multiple_of(i * _BS, _BS)
    o_ref[0, 0:1, pl.ds(off, _BS)] = (scale * jnp.exp2(delta)) * acc

    @pl.when(i == nsb - 1)
    def _():
        row = o_ref[0]
        total = jnp.sum(row, axis=1, keepdims=True)
        denom = jnp.maximum(total, 1e-5)
        o_ref[0] = row * (1.0 / denom)


def kernel(T, S):
    B, nv, _ = T.shape
    ns = S.shape[0]
    d = 3
    c = (4.0 / (d + 2)) ** (1.0 / (d + 4)) * float(nv) ** (-1.0 / (d + 4))
    cf = (2.0 * math.pi) ** (-1.5) / c ** 3
    cn = -1.0 / (2.0 * c ** 2)

    st = S.T

    body = functools.partial(_histogram_kernel, nv=nv, ns=ns, cn=cn, cf=cf)
    out = pl.pallas_call(
        body,
        grid=(B, ns // _BS),
        in_specs=[
            pl.BlockSpec((1, nv, 3), lambda b, i: (b, 0, 0)),
            pl.BlockSpec((3, _BS), lambda b, i: (0, i)),
        ],
        out_specs=pl.BlockSpec((1, 1, ns), lambda b, i: (b, 0, 0)),
        out_shape=jax.ShapeDtypeStruct((B, 1, ns), jnp.float32),
        scratch_shapes=[
            pltpu.SMEM((1, 1), jnp.float32),
            pltpu.VMEM((nv, 5), jnp.float32),
        ],
        compiler_params=pltpu.CompilerParams(
            dimension_semantics=("parallel", "arbitrary"),
            vmem_limit_bytes=56 * 1024 * 1024,
        ),
    )(T, st)
    return out.reshape(B, ns)
